# Initial kernel scaffold; baseline (speedup 1.0000x reference)
import jax
import jax.numpy as jnp
from jax.experimental import pallas as pl


def kernel(llr, betas, alphas):
    raise NotImplementedError("write your pallas kernel here")


# speedup vs baseline: 10005.8945x; 10005.8945x over previous
import jax
import jax.numpy as jnp
from jax.experimental import pallas as pl
from jax.experimental.pallas import tpu as pltpu

N = 65536
M = 32768
DV = 4
DC = 8
T = 8
W = N // DC


def _decode_kernel(betas_ref, alphas_ref, llr_ref, dec_ref, post_ref):
    llr = llr_ref[...]
    llr3 = llr[None, :, :]
    v2c = jnp.broadcast_to(llr3, (DV, DC, W))
    big = jnp.float32(1e30)
    c2v = jnp.zeros((DV, DC, W), dtype=jnp.float32)
    for t in range(T):
        beta = betas_ref[t]
        alpha = alphas_ref[t]
        mag = jnp.abs(v2c)
        sgn = jnp.sign(v2c)
        total_sign = sgn[:, 0, :]
        for k in range(1, DC):
            total_sign = total_sign * sgn[:, k, :]
        total_sign = total_sign[:, None, :]
        ext_sign = total_sign * sgn
        min1 = jnp.min(mag, axis=1, keepdims=True)
        is_min = mag <= min1
        min2 = jnp.min(jnp.where(is_min, big, mag), axis=1, keepdims=True)
        ext_mag = jnp.where(is_min, min2, min1)
        c2v = beta * ext_mag * ext_sign
        sum_c2v = jnp.sum(c2v, axis=0, keepdims=True)
        v2c = llr3 + alpha * (sum_c2v - c2v)
    post = llr + jnp.sum(c2v, axis=0)
    post_ref[...] = post
    dec_ref[...] = (post < 0).astype(jnp.int32)


def kernel(llr, betas, alphas):
    llr2 = llr.reshape(DC, W)
    dec2, post2 = pl.pallas_call(
        _decode_kernel,
        out_shape=(
            jax.ShapeDtypeStruct((DC, W), jnp.int32),
            jax.ShapeDtypeStruct((DC, W), jnp.float32),
        ),
        in_specs=[
            pl.BlockSpec(memory_space=pltpu.SMEM),
            pl.BlockSpec(memory_space=pltpu.SMEM),
            pl.BlockSpec(memory_space=pltpu.VMEM),
        ],
        out_specs=(
            pl.BlockSpec(memory_space=pltpu.VMEM),
            pl.BlockSpec(memory_space=pltpu.VMEM),
        ),
    )(betas, alphas, llr2)
    return dec2.reshape(N), post2.reshape(N)
